# baseline (speedup 1.0000x reference)
import jax
import jax.numpy as jnp
from jax import lax
from jax.experimental import pallas as pl
from jax.experimental.pallas import tpu as pltpu

TOKENS = 32768
HIDDEN = 2048
FEAT_LEN = 1024
NUM_FEATS = 4
COPY_BLOCK = 1024
FB = 128
NTILE = FEAT_LEN // FB + 1


def _copy_body(ord_ref, emb_ref, out_ref):
    out_ref[...] = emb_ref[...]


CB = 256
NCB = TOKENS // CB
SKIP_STEP = 4


def _make_copy(n):
    grid_spec = pltpu.PrefetchScalarGridSpec(
        num_scalar_prefetch=1,
        grid=(n,),
        in_specs=[pl.BlockSpec((CB, HIDDEN), lambda c, ord_ref: (ord_ref[c], 0))],
        out_specs=pl.BlockSpec((CB, HIDDEN), lambda c, ord_ref: (ord_ref[c], 0)),
    )

    def run(order, emb):
        return pl.pallas_call(
            _copy_body,
            grid_spec=grid_spec,
            out_shape=jax.ShapeDtypeStruct((TOKENS, HIDDEN), jnp.float32),
        )(order, emb)

    return run


def _feat_body(i, locs_s, fb_ref, cur_hbm, out_ref, prev, e0, e1):
    t = pl.program_id(0)
    loc = locs_s[i]
    base = pl.multiple_of((loc // FB) * FB, FB)
    r = loc - base

    @pl.when(t == 0)
    def _():
        pltpu.sync_copy(cur_hbm.at[pl.ds(base, FB)], e0)
        pltpu.sync_copy(cur_hbm.at[pl.ds(base + FEAT_LEN, FB)], e1)

    fb = fb_ref[...]
    fa = jnp.where(t == 0, fb, prev[...])
    u = jnp.concatenate([fa, fb], axis=0)
    v = pltpu.roll(u, FB + r, 0)[:FB, :]
    g = base + FB * t + lax.broadcasted_iota(jnp.int32, (FB, 1), 0)
    infeat = (g >= loc) & (g < loc + FEAT_LEN)
    cur_tile = jnp.where(t == 0, e0[...], e1[...])
    out_ref[...] = jnp.where(infeat, v, cur_tile)
    prev[...] = fb


def kernel(embeddings, feature_0, feature_1, feature_2, feature_3, multimodal_locs):
    locs = multimodal_locs.astype(jnp.int32)

    ends = locs + FEAT_LEN
    reach = [None] * NUM_FEATS
    reach[3] = ends[3]
    for i in (2, 1, 0):
        reach[i] = jnp.where(
            locs[i + 1] <= ends[i], jnp.maximum(ends[i], reach[i + 1]), ends[i]
        )
    reach_v = jnp.stack(reach)
    blk_start = jnp.arange(NCB, dtype=jnp.int32) * CB
    covered = jnp.any(
        (locs[None, :] <= blk_start[:, None])
        & (blk_start[:, None] + CB <= reach_v[None, :]),
        axis=1,
    )
    order = jnp.argsort(covered, stable=True).astype(jnp.int32)
    n_cov = jnp.sum(covered.astype(jnp.int32))
    v = jnp.minimum(n_cov // SKIP_STEP, NCB // 16).astype(jnp.int32)
    branches = [
        _make_copy(NCB - k * SKIP_STEP) for k in range(NCB // 16 + 1)
    ]
    out = lax.switch(v, branches, order, embeddings)

    feats = [feature_0, feature_1, feature_2, feature_3]
    for i in range(NUM_FEATS):
        grid_spec = pltpu.PrefetchScalarGridSpec(
            num_scalar_prefetch=1,
            grid=(NTILE,),
            in_specs=[
                pl.BlockSpec(
                    (FB, HIDDEN),
                    lambda t, locs_ref: (jnp.minimum(t, FEAT_LEN // FB - 1), 0),
                ),
                pl.BlockSpec(memory_space=pltpu.MemorySpace.HBM),
            ],
            out_specs=pl.BlockSpec(
                (FB, HIDDEN),
                lambda t, locs_ref, i=i: (locs_ref[i] // FB + t, 0),
            ),
            scratch_shapes=[
                pltpu.VMEM((FB, HIDDEN), jnp.float32),
                pltpu.VMEM((FB, HIDDEN), jnp.float32),
                pltpu.VMEM((FB, HIDDEN), jnp.float32),
            ],
        )
        out = pl.pallas_call(
            lambda *a, i=i: _feat_body(i, *a),
            grid_spec=grid_spec,
            out_shape=jax.ShapeDtypeStruct((TOKENS, HIDDEN), jnp.float32),
            input_output_aliases={2: 0},
        )(locs, feats[i], out)
    return out


# speedup vs baseline: 1.0109x; 1.0109x over previous
import jax
import jax.numpy as jnp
from jax import lax
from jax.experimental import pallas as pl
from jax.experimental.pallas import tpu as pltpu

TOKENS = 32768
HIDDEN = 2048
FEAT_LEN = 1024
NUM_FEATS = 4
COPY_BLOCK = 1024
FB = 128
NTILE = FEAT_LEN // FB + 1


def _copy_body(ord_ref, emb_ref, out_ref):
    out_ref[...] = emb_ref[...]


CB = 1024
NCB = TOKENS // CB
SKIP_STEP = 1


def _make_copy(n):
    grid_spec = pltpu.PrefetchScalarGridSpec(
        num_scalar_prefetch=1,
        grid=(n,),
        in_specs=[pl.BlockSpec((CB, HIDDEN), lambda c, ord_ref: (ord_ref[c], 0))],
        out_specs=pl.BlockSpec((CB, HIDDEN), lambda c, ord_ref: (ord_ref[c], 0)),
    )

    def run(order, emb):
        return pl.pallas_call(
            _copy_body,
            grid_spec=grid_spec,
            out_shape=jax.ShapeDtypeStruct((TOKENS, HIDDEN), jnp.float32),
        )(order, emb)

    return run


def _feat_body(i, locs_s, fb_ref, cur_hbm, out_ref, prev, e0, e1):
    t = pl.program_id(0)
    loc = locs_s[i]
    base = pl.multiple_of((loc // FB) * FB, FB)
    r = loc - base

    @pl.when(t == 0)
    def _():
        pltpu.sync_copy(cur_hbm.at[pl.ds(base, FB)], e0)
        pltpu.sync_copy(cur_hbm.at[pl.ds(base + FEAT_LEN, FB)], e1)

    fb = fb_ref[...]
    fa = jnp.where(t == 0, fb, prev[...])
    u = jnp.concatenate([fa, fb], axis=0)
    v = pltpu.roll(u, FB + r, 0)[:FB, :]
    g = base + FB * t + lax.broadcasted_iota(jnp.int32, (FB, 1), 0)
    infeat = (g >= loc) & (g < loc + FEAT_LEN)
    cur_tile = jnp.where(t == 0, e0[...], e1[...])
    out_ref[...] = jnp.where(infeat, v, cur_tile)
    prev[...] = fb


def kernel(embeddings, feature_0, feature_1, feature_2, feature_3, multimodal_locs):
    locs = multimodal_locs.astype(jnp.int32)

    ends = locs + FEAT_LEN
    reach = [None] * NUM_FEATS
    reach[3] = ends[3]
    for i in (2, 1, 0):
        reach[i] = jnp.where(
            locs[i + 1] <= ends[i], jnp.maximum(ends[i], reach[i + 1]), ends[i]
        )
    reach_v = jnp.stack(reach)
    blk_start = jnp.arange(NCB, dtype=jnp.int32) * CB
    covered = jnp.any(
        (locs[None, :] <= blk_start[:, None])
        & (blk_start[:, None] + CB <= reach_v[None, :]),
        axis=1,
    )
    order = jnp.argsort(covered, stable=True).astype(jnp.int32)
    n_cov = jnp.sum(covered.astype(jnp.int32))
    v = jnp.minimum(n_cov // SKIP_STEP, 3).astype(jnp.int32)
    branches = [_make_copy(NCB - k * SKIP_STEP) for k in range(4)]
    out = lax.switch(v, branches, order, embeddings)

    feats = [feature_0, feature_1, feature_2, feature_3]
    for i in range(NUM_FEATS):
        grid_spec = pltpu.PrefetchScalarGridSpec(
            num_scalar_prefetch=1,
            grid=(NTILE,),
            in_specs=[
                pl.BlockSpec(
                    (FB, HIDDEN),
                    lambda t, locs_ref: (jnp.minimum(t, FEAT_LEN // FB - 1), 0),
                ),
                pl.BlockSpec(memory_space=pltpu.MemorySpace.HBM),
            ],
            out_specs=pl.BlockSpec(
                (FB, HIDDEN),
                lambda t, locs_ref, i=i: (locs_ref[i] // FB + t, 0),
            ),
            scratch_shapes=[
                pltpu.VMEM((FB, HIDDEN), jnp.float32),
                pltpu.VMEM((FB, HIDDEN), jnp.float32),
                pltpu.VMEM((FB, HIDDEN), jnp.float32),
            ],
        )
        out = pl.pallas_call(
            lambda *a, i=i: _feat_body(i, *a),
            grid_spec=grid_spec,
            out_shape=jax.ShapeDtypeStruct((TOKENS, HIDDEN), jnp.float32),
            input_output_aliases={2: 0},
        )(locs, feats[i], out)
    return out


# speedup vs baseline: 1.0788x; 1.0672x over previous
import jax
import jax.numpy as jnp
from jax import lax
from jax.experimental import pallas as pl
from jax.experimental.pallas import tpu as pltpu

TOKENS = 32768
HIDDEN = 2048
FEAT_LEN = 1024
NUM_FEATS = 4
COPY_BLOCK = 1024
FB = 128
NTILE = FEAT_LEN // FB + 1


def _plain_copy_body(emb_ref, out_ref):
    out_ref[...] = emb_ref[...]


def _feat_body(i, locs_s, fb_ref, cur_hbm, out_ref, prev, e0, e1):
    t = pl.program_id(0)
    loc = locs_s[i]
    base = pl.multiple_of((loc // FB) * FB, FB)
    r = loc - base

    @pl.when(t == 0)
    def _():
        pltpu.sync_copy(cur_hbm.at[pl.ds(base, FB)], e0)
        pltpu.sync_copy(cur_hbm.at[pl.ds(base + FEAT_LEN, FB)], e1)

    fb = fb_ref[...]
    fa = jnp.where(t == 0, fb, prev[...])
    u = jnp.concatenate([fa, fb], axis=0)
    v = pltpu.roll(u, FB + r, 0)[:FB, :]
    g = base + FB * t + lax.broadcasted_iota(jnp.int32, (FB, 1), 0)
    infeat = (g >= loc) & (g < loc + FEAT_LEN)
    cur_tile = jnp.where(t == 0, e0[...], e1[...])
    out_ref[...] = jnp.where(infeat, v, cur_tile)
    prev[...] = fb


def kernel(embeddings, feature_0, feature_1, feature_2, feature_3, multimodal_locs):
    locs = multimodal_locs.astype(jnp.int32)

    out = pl.pallas_call(
        _plain_copy_body,
        grid=(TOKENS // COPY_BLOCK,),
        in_specs=[pl.BlockSpec((COPY_BLOCK, HIDDEN), lambda c: (c, 0))],
        out_specs=pl.BlockSpec((COPY_BLOCK, HIDDEN), lambda c: (c, 0)),
        out_shape=jax.ShapeDtypeStruct((TOKENS, HIDDEN), jnp.float32),
    )(embeddings)

    feats = [feature_0, feature_1, feature_2, feature_3]
    for i in range(NUM_FEATS):
        grid_spec = pltpu.PrefetchScalarGridSpec(
            num_scalar_prefetch=1,
            grid=(NTILE,),
            in_specs=[
                pl.BlockSpec(
                    (FB, HIDDEN),
                    lambda t, locs_ref: (jnp.minimum(t, FEAT_LEN // FB - 1), 0),
                ),
                pl.BlockSpec(memory_space=pltpu.MemorySpace.HBM),
            ],
            out_specs=pl.BlockSpec(
                (FB, HIDDEN),
                lambda t, locs_ref, i=i: (locs_ref[i] // FB + t, 0),
            ),
            scratch_shapes=[
                pltpu.VMEM((FB, HIDDEN), jnp.float32),
                pltpu.VMEM((FB, HIDDEN), jnp.float32),
                pltpu.VMEM((FB, HIDDEN), jnp.float32),
            ],
        )
        out = pl.pallas_call(
            lambda *a, i=i: _feat_body(i, *a),
            grid_spec=grid_spec,
            out_shape=jax.ShapeDtypeStruct((TOKENS, HIDDEN), jnp.float32),
            input_output_aliases={2: 0},
        )(locs, feats[i], out)
    return out
